# baseline (speedup 1.0000x reference)
import functools

import jax
import jax.numpy as jnp
from jax import lax
from jax.experimental import pallas as pl
from jax.experimental.pallas import tpu as pltpu
from jax.experimental.pallas import tpu_sc as plsc

_S = 8
_L = 32
_M = 8
_B = _S * _L
_D = 128
_SENT = jnp.iinfo(jnp.int32).max

_NC = 2
_NS = 16
_NW = _NC * _NS
_BPW = _B // _NW


@functools.cache
def _sc_gather_fn():
    mesh = plsc.VectorSubcoreMesh(
        core_axis_name="c", subcore_axis_name="s",
        num_cores=_NC, num_subcores=_NS)

    @functools.partial(
        pl.kernel,
        mesh=mesh,
        out_type=jax.ShapeDtypeStruct((_B, _D), jnp.float32),
        scratch_types=[
            pltpu.VMEM((_BPW,), jnp.int32),
            pltpu.VMEM((_BPW, _D), jnp.float32),
            pltpu.SemaphoreType.DMA,
        ],
    )
    def _sc_gather(emb_hbm, idx_hbm, out_hbm, idx_v, rows_v, sem):
        wid = lax.axis_index("s") * _NC + lax.axis_index("c")
        base = wid * _BPW
        pltpu.sync_copy(idx_hbm.at[pl.ds(base, _BPW)], idx_v)
        pltpu.async_copy(emb_hbm.at[idx_v], rows_v, sem).wait()
        pltpu.sync_copy(rows_v, out_hbm.at[pl.ds(base, _BPW)])

    return _sc_gather


def _match_body(bins_ref, thr_ref, table_ref, ba_ref, mv_ref,
                glob_ref, loc_ref):
    i32 = jnp.int32
    f32 = jnp.float32
    inf = jnp.float32(jnp.inf)

    bi = lax.broadcasted_iota(i32, (_B, _M), 0)
    ji = lax.broadcasted_iota(i32, (_B, _M), 1)
    glob_ref[...] = jnp.full((_B, _M), _SENT, i32)
    loc_ref[...] = jnp.where((ji == 0) & (bi < _L), bi, 0)

    def _init0(i, carry):
        v = bins_ref[0, i]
        glob_ref[pl.ds(i, 1), pl.ds(0, 1)] = v.reshape(1, 1)
        return carry

    lax.fori_loop(0, _L, _init0, 0)

    tab = table_ref[...]
    tabT = jnp.swapaxes(tab, 0, 1)

    def _comp(t):
        acc = None
        for i in range(_D // 8):
            f = 8 * i + t
            d = tab[_L:, f:f + 1] - tabT[f:f + 1, :]
            sq = d * d
            acc = sq if acc is None else acc + sq
        return acc

    s01 = _comp(0) + _comp(1)
    s23 = _comp(2) + _comp(3)
    s45 = _comp(4) + _comp(5)
    s67 = _comp(6) + _comp(7)
    dfull = jnp.sqrt((s01 + s23) + (s45 + s67))

    lane32 = lax.broadcasted_iota(i32, (1, _L), 1)
    lane8 = lax.broadcasted_iota(i32, (1, _M), 1)
    kiotaS = lax.broadcasted_iota(i32, (_B, _B), 0)
    bioT = lax.broadcasted_iota(i32, (_L, _B), 1)
    cioT = lax.broadcasted_iota(i32, (_L, _B), 0)
    flatT = bioT * _L + cioT
    thr = thr_ref[0, 0]

    ba_ref[pl.ds(0, 1), :] = lane32
    nbins = jnp.int32(_L)

    for cand in range(1, _S):
        dcT = dfull[_L * (cand - 1):_L * cand, :]
        locT = jnp.swapaxes(loc_ref[...], 0, 1)
        globT = jnp.swapaxes(glob_ref[...], 0, 1)
        cntT = jnp.sum((globT != _SENT).astype(i32), axis=0,
                       keepdims=True)

        ohsT = jnp.concatenate(
            [(kiotaS == locT[j:j + 1, :]).astype(f32) for j in range(_M)],
            axis=1)
        dallT = lax.dot_general(dcT, ohsT, (((1,), (0,)), ((), ())),
                                precision=lax.Precision.HIGHEST,
                                preferred_element_type=f32)
        dms = [jnp.where(cntT > j, dallT[:, _B * j:_B * (j + 1)],
                         jnp.float32(0.0)) for j in range(_M)]
        seq = dms[0]
        for j in range(1, _M):
            seq = seq + dms[j]
        tree = ((dms[0] + dms[1]) + (dms[2] + dms[3])) + (
            (dms[4] + dms[5]) + (dms[6] + dms[7]))
        ssum = jnp.where(cntT == _M, tree, seq)
        mean = ssum / jnp.maximum(cntT, 1).astype(f32)
        mat = jnp.where(bioT < nbins, mean, inf)

        def _step(k, carry):
            masked, vsv, prev_m, nbins_c, ba_row = carry
            m = jnp.min(masked)
            idx = jnp.min(jnp.where(masked == m, flatT, jnp.int32(2 ** 30)))
            r = idx // _L
            c = idx - r * _L
            vsv = jnp.where(lane32 == k, m, vsv)
            masked = jnp.where((bioT == r) | (cioT == c), inf, masked)
            nb = -prev_m
            ssm = m + nb
            bv = ssm - m
            av = ssm - bv
            e = (m - av) + (nb - bv)
            exceeds = (ssm > thr) | ((ssm == thr) & (e > jnp.float32(0.0)))
            f = (k == 0) | jnp.logical_not(exceeds)
            tb = jnp.where(f, r, nbins_c)
            v = bins_ref[cand, c]
            rowg = glob_ref[pl.ds(tb, 1), :]
            rowl = loc_ref[pl.ds(tb, 1), :]
            present = jnp.sum((rowg == v).astype(i32)) > 0
            p = jnp.sum((rowg < v).astype(i32))
            shg = jnp.concatenate([rowg[:, :1], rowg[:, :_M - 1]], axis=1)
            shl = jnp.concatenate([rowl[:, :1], rowl[:, :_M - 1]], axis=1)
            newg = jnp.where(lane8 < p, rowg, jnp.where(lane8 == p, v, shg))
            newloc = jnp.int32(cand * _L) + c
            newl = jnp.where(lane8 < p, rowl,
                             jnp.where(lane8 == p, newloc, shl))
            glob_ref[pl.ds(tb, 1), :] = jnp.where(present, rowg, newg)
            loc_ref[pl.ds(tb, 1), :] = jnp.where(present, rowl, newl)
            ba_row = jnp.where(lane32 == c, tb, ba_row)
            nbins_c = nbins_c + jnp.where(f, 0, 1).astype(i32)
            return masked, vsv, m, nbins_c, ba_row

        carry = (mat, jnp.zeros((1, _L), f32),
                 jnp.float32(0.0), nbins, jnp.full((1, _L), -1, i32))
        for k in range(_L):
            carry = _step(k, carry)
        _, vsv, _, nbins, ba_row = carry

        mv_ref[pl.ds(cand - 1, 1), :] = vsv
        ba_ref[pl.ds(cand, 1), :] = ba_row


def kernel(binSets, embeds, THRESHOLD):
    bins = jnp.asarray(binSets, jnp.int32)
    emb = jnp.asarray(embeds, jnp.float32)
    thr = jnp.asarray(THRESHOLD).astype(jnp.float32).reshape(1, 1)
    table = _sc_gather_fn()(emb, bins.reshape(-1))
    ba, mv = pl.pallas_call(
        _match_body,
        in_specs=[
            pl.BlockSpec(memory_space=pltpu.SMEM),
            pl.BlockSpec(memory_space=pltpu.SMEM),
            pl.BlockSpec(memory_space=pltpu.VMEM),
        ],
        out_specs=[
            pl.BlockSpec(memory_space=pltpu.VMEM),
            pl.BlockSpec(memory_space=pltpu.VMEM),
        ],
        out_shape=[
            jax.ShapeDtypeStruct((_S, _L), jnp.int32),
            jax.ShapeDtypeStruct((_S - 1, _L), jnp.float32),
        ],
        scratch_shapes=[
            pltpu.VMEM((_B, _M), jnp.int32),
            pltpu.VMEM((_B, _M), jnp.int32),
        ],
    )(bins, thr, table)
    return ba, jnp.reshape(mv, (-1,))


# speedup vs baseline: 1.0347x; 1.0347x over previous
import functools

import jax
import jax.numpy as jnp
from jax import lax
from jax.experimental import pallas as pl
from jax.experimental.pallas import tpu as pltpu
from jax.experimental.pallas import tpu_sc as plsc

_S = 8
_L = 32
_M = 8
_B = _S * _L
_D = 128
_SENT = jnp.iinfo(jnp.int32).max

_NC = 2
_NS = 16
_NW = _NC * _NS
_BPW = _B // _NW


@functools.cache
def _sc_gather_fn():
    mesh = plsc.VectorSubcoreMesh(
        core_axis_name="c", subcore_axis_name="s",
        num_cores=_NC, num_subcores=_NS)

    @functools.partial(
        pl.kernel,
        mesh=mesh,
        out_type=jax.ShapeDtypeStruct((_B, _D), jnp.float32),
        scratch_types=[
            pltpu.VMEM((_BPW,), jnp.int32),
            pltpu.VMEM((_BPW, _D), jnp.float32),
            pltpu.SemaphoreType.DMA,
        ],
    )
    def _sc_gather(emb_hbm, idx_hbm, out_hbm, idx_v, rows_v, sem):
        wid = lax.axis_index("s") * _NC + lax.axis_index("c")
        base = wid * _BPW
        pltpu.sync_copy(idx_hbm.at[pl.ds(base, _BPW)], idx_v)
        pltpu.async_copy(emb_hbm.at[idx_v], rows_v, sem).wait()
        pltpu.sync_copy(rows_v, out_hbm.at[pl.ds(base, _BPW)])

    return _sc_gather


def _match_body(bins_ref, thr_ref, table_ref, ba_ref, mv_ref,
                glob_ref, loc_ref):
    i32 = jnp.int32
    f32 = jnp.float32
    inf = jnp.float32(jnp.inf)

    bi = lax.broadcasted_iota(i32, (_B, _M), 0)
    ji = lax.broadcasted_iota(i32, (_B, _M), 1)
    glob_ref[...] = jnp.full((_B, _M), _SENT, i32)
    loc_ref[...] = jnp.where((ji == 0) & (bi < _L), bi, 0)

    def _init0(i, carry):
        v = bins_ref[0, i]
        glob_ref[pl.ds(i, 1), pl.ds(0, 1)] = v.reshape(1, 1)
        return carry

    lax.fori_loop(0, _L, _init0, 0)

    tab = table_ref[...]
    tabT = jnp.swapaxes(tab, 0, 1)

    def _comp(t):
        acc = None
        for i in range(_D // 8):
            f = 8 * i + t
            d = tab[_L:, f:f + 1] - tabT[f:f + 1, :]
            sq = d * d
            acc = sq if acc is None else acc + sq
        return acc

    s01 = _comp(0) + _comp(1)
    s23 = _comp(2) + _comp(3)
    s45 = _comp(4) + _comp(5)
    s67 = _comp(6) + _comp(7)
    dfull = jnp.sqrt((s01 + s23) + (s45 + s67))

    lane32 = lax.broadcasted_iota(i32, (1, _L), 1)
    lane8 = lax.broadcasted_iota(i32, (1, _M), 1)
    kiotaS = lax.broadcasted_iota(i32, (_B, _B), 0)
    bioT = lax.broadcasted_iota(i32, (_L, _B), 1)
    cioT = lax.broadcasted_iota(i32, (_L, _B), 0)
    flatT = bioT * _L + cioT
    thr = thr_ref[0, 0]

    ba_ref[pl.ds(0, 1), :] = lane32
    nbins = jnp.int32(_L)

    for cand in range(1, _S):
        dcT = dfull[_L * (cand - 1):_L * cand, :]
        locT = jnp.swapaxes(loc_ref[...], 0, 1)
        globT = jnp.swapaxes(glob_ref[...], 0, 1)
        cntT = jnp.sum((globT != _SENT).astype(i32), axis=0,
                       keepdims=True)

        ohsT = jnp.concatenate(
            [(kiotaS == locT[j:j + 1, :]).astype(f32) for j in range(_M)],
            axis=1)
        dallT = lax.dot_general(dcT, ohsT, (((1,), (0,)), ((), ())),
                                precision=lax.Precision.HIGHEST,
                                preferred_element_type=f32)
        dms = [jnp.where(cntT > j, dallT[:, _B * j:_B * (j + 1)],
                         jnp.float32(0.0)) for j in range(_M)]
        seq = dms[0]
        for j in range(1, _M):
            seq = seq + dms[j]
        tree = ((dms[0] + dms[1]) + (dms[2] + dms[3])) + (
            (dms[4] + dms[5]) + (dms[6] + dms[7]))
        ssum = jnp.where(cntT == _M, tree, seq)
        mean = ssum / jnp.maximum(cntT, 1).astype(f32)
        mat = jnp.where(bioT < nbins, mean, inf)

        def _step(k, carry):
            masked, vsv, prev_m, nbins_c, ba_row = carry
            m = jnp.min(masked)
            idx = jnp.min(jnp.where(masked == m, flatT, jnp.int32(2 ** 30)))
            r = lax.shift_right_logical(idx, 5)
            c = lax.bitwise_and(idx, jnp.int32(_L - 1))
            vsv = jnp.where(lane32 == k, m, vsv)
            masked = jnp.where((bioT == r) | (cioT == c), inf, masked)
            nb = -prev_m
            ssm = m + nb
            bv = ssm - m
            av = ssm - bv
            e = (m - av) + (nb - bv)
            exceeds = (ssm > thr) | ((ssm == thr) & (e > jnp.float32(0.0)))
            f = (k == 0) | jnp.logical_not(exceeds)
            tb = jnp.where(f, r, nbins_c)
            v = bins_ref[cand, c]
            rowg = glob_ref[pl.ds(tb, 1), :]
            rowl = loc_ref[pl.ds(tb, 1), :]
            present = jnp.sum((rowg == v).astype(i32)) > 0
            p = jnp.sum((rowg < v).astype(i32))
            shg = jnp.concatenate([rowg[:, :1], rowg[:, :_M - 1]], axis=1)
            shl = jnp.concatenate([rowl[:, :1], rowl[:, :_M - 1]], axis=1)
            newg = jnp.where(lane8 < p, rowg, jnp.where(lane8 == p, v, shg))
            newloc = jnp.int32(cand * _L) + c
            newl = jnp.where(lane8 < p, rowl,
                             jnp.where(lane8 == p, newloc, shl))
            glob_ref[pl.ds(tb, 1), :] = jnp.where(present, rowg, newg)
            loc_ref[pl.ds(tb, 1), :] = jnp.where(present, rowl, newl)
            ba_row = jnp.where(lane32 == c, tb, ba_row)
            nbins_c = nbins_c + jnp.where(f, 0, 1).astype(i32)
            return masked, vsv, m, nbins_c, ba_row

        def _step8(k8, carry):
            for t in range(8):
                carry = _step(k8 * 8 + t, carry)
            return carry

        carry0 = (mat, jnp.zeros((1, _L), f32),
                  jnp.float32(0.0), nbins, jnp.full((1, _L), -1, i32))
        _, vsv, _, nbins, ba_row = lax.fori_loop(0, _L // 8, _step8, carry0)

        mv_ref[pl.ds(cand - 1, 1), :] = vsv
        ba_ref[pl.ds(cand, 1), :] = ba_row


def kernel(binSets, embeds, THRESHOLD):
    bins = jnp.asarray(binSets, jnp.int32)
    emb = jnp.asarray(embeds, jnp.float32)
    thr = jnp.asarray(THRESHOLD).astype(jnp.float32).reshape(1, 1)
    table = _sc_gather_fn()(emb, bins.reshape(-1))
    ba, mv = pl.pallas_call(
        _match_body,
        in_specs=[
            pl.BlockSpec(memory_space=pltpu.SMEM),
            pl.BlockSpec(memory_space=pltpu.SMEM),
            pl.BlockSpec(memory_space=pltpu.VMEM),
        ],
        out_specs=[
            pl.BlockSpec(memory_space=pltpu.VMEM),
            pl.BlockSpec(memory_space=pltpu.VMEM),
        ],
        out_shape=[
            jax.ShapeDtypeStruct((_S, _L), jnp.int32),
            jax.ShapeDtypeStruct((_S - 1, _L), jnp.float32),
        ],
        scratch_shapes=[
            pltpu.VMEM((_B, _M), jnp.int32),
            pltpu.VMEM((_B, _M), jnp.int32),
        ],
    )(bins, thr, table)
    return ba, jnp.reshape(mv, (-1,))


# speedup vs baseline: 1.0464x; 1.0113x over previous
import functools

import jax
import jax.numpy as jnp
from jax import lax
from jax.experimental import pallas as pl
from jax.experimental.pallas import tpu as pltpu
from jax.experimental.pallas import tpu_sc as plsc

_S = 8
_L = 32
_M = 8
_B = _S * _L
_D = 128
_SENT = jnp.iinfo(jnp.int32).max

_NC = 2
_NS = 16
_NW = _NC * _NS
_BPW = _B // _NW


@functools.cache
def _sc_gather_fn():
    mesh = plsc.VectorSubcoreMesh(
        core_axis_name="c", subcore_axis_name="s",
        num_cores=_NC, num_subcores=_NS)

    @functools.partial(
        pl.kernel,
        mesh=mesh,
        out_type=jax.ShapeDtypeStruct((_B, _D), jnp.float32),
        scratch_types=[
            pltpu.VMEM((_BPW,), jnp.int32),
            pltpu.VMEM((_BPW, _D), jnp.float32),
            pltpu.SemaphoreType.DMA,
        ],
    )
    def _sc_gather(emb_hbm, idx_hbm, out_hbm, idx_v, rows_v, sem):
        wid = lax.axis_index("s") * _NC + lax.axis_index("c")
        base = wid * _BPW
        pltpu.sync_copy(idx_hbm.at[pl.ds(base, _BPW)], idx_v)
        pltpu.async_copy(emb_hbm.at[idx_v], rows_v, sem).wait()
        pltpu.sync_copy(rows_v, out_hbm.at[pl.ds(base, _BPW)])

    return _sc_gather


def _match_body(bins_ref, thr_ref, table_ref, ba_ref, mv_ref,
                glob_ref, loc_ref):
    i32 = jnp.int32
    f32 = jnp.float32
    inf = jnp.float32(jnp.inf)

    bi = lax.broadcasted_iota(i32, (_B, _M), 0)
    ji = lax.broadcasted_iota(i32, (_B, _M), 1)
    glob_ref[...] = jnp.full((_B, _M), _SENT, i32)
    loc_ref[...] = jnp.where((ji == 0) & (bi < _L), bi, 0)

    def _init0(i, carry):
        v = bins_ref[0, i]
        glob_ref[pl.ds(i, 1), pl.ds(0, 1)] = v.reshape(1, 1)
        return carry

    lax.fori_loop(0, _L, _init0, 0)

    tab = table_ref[...]
    tabT = jnp.swapaxes(tab, 0, 1)

    def _comp(t):
        acc = None
        for i in range(_D // 8):
            f = 8 * i + t
            d = tab[_L:, f:f + 1] - tabT[f:f + 1, :]
            sq = d * d
            acc = sq if acc is None else acc + sq
        return acc

    s01 = _comp(0) + _comp(1)
    s23 = _comp(2) + _comp(3)
    s45 = _comp(4) + _comp(5)
    s67 = _comp(6) + _comp(7)
    dfull = jnp.sqrt((s01 + s23) + (s45 + s67))

    lane32 = lax.broadcasted_iota(i32, (1, _L), 1)
    lane8 = lax.broadcasted_iota(i32, (1, _M), 1)
    kiotaS = lax.broadcasted_iota(i32, (_B, _B), 0)
    bioT = lax.broadcasted_iota(i32, (_L, _B), 1)
    cioT = lax.broadcasted_iota(i32, (_L, _B), 0)
    flatT = bioT * _L + cioT
    thr = thr_ref[0, 0]

    ba_ref[pl.ds(0, 1), :] = lane32
    nbins = jnp.int32(_L)

    for cand in range(1, _S):
        dcT = dfull[_L * (cand - 1):_L * cand, :]
        locT = jnp.swapaxes(loc_ref[...], 0, 1)
        globT = jnp.swapaxes(glob_ref[...], 0, 1)
        cntT = jnp.sum((globT != _SENT).astype(i32), axis=0,
                       keepdims=True)

        ohsT = jnp.concatenate(
            [(kiotaS == locT[j:j + 1, :]).astype(f32) for j in range(_M)],
            axis=1)
        dallT = lax.dot_general(dcT, ohsT, (((1,), (0,)), ((), ())),
                                precision=lax.Precision.HIGHEST,
                                preferred_element_type=f32)
        dms = [jnp.where(cntT > j, dallT[:, _B * j:_B * (j + 1)],
                         jnp.float32(0.0)) for j in range(_M)]
        seq = dms[0]
        for j in range(1, _M):
            seq = seq + dms[j]
        tree = ((dms[0] + dms[1]) + (dms[2] + dms[3])) + (
            (dms[4] + dms[5]) + (dms[6] + dms[7]))
        ssum = jnp.where(cntT == _M, tree, seq)
        mean = ssum / jnp.maximum(cntT, 1).astype(f32)
        mat = jnp.where(bioT < nbins, mean, inf)

        def _step(k, carry):
            masked, vsv, prev_m, nbins_c, ba_row = carry
            m = jnp.min(masked)
            idx = jnp.min(jnp.where(masked == m, flatT, jnp.int32(2 ** 30)))
            r = lax.shift_right_logical(idx, 5)
            c = lax.bitwise_and(idx, jnp.int32(_L - 1))
            vsv = jnp.where(lane32 == k, m, vsv)
            masked = jnp.where((bioT == r) | (cioT == c), inf, masked)
            nb = -prev_m
            ssm = m + nb
            bv = ssm - m
            av = ssm - bv
            e = (m - av) + (nb - bv)
            exceeds = (ssm > thr) | ((ssm == thr) & (e > jnp.float32(0.0)))
            f = (k == 0) | jnp.logical_not(exceeds)
            tb = jnp.where(f, r, nbins_c)
            v = bins_ref[cand, c]
            rowg = glob_ref[pl.ds(tb, 1), :]
            rowl = loc_ref[pl.ds(tb, 1), :]
            present = jnp.sum((rowg == v).astype(i32)) > 0
            p = jnp.sum((rowg < v).astype(i32))
            shg = jnp.concatenate([rowg[:, :1], rowg[:, :_M - 1]], axis=1)
            shl = jnp.concatenate([rowl[:, :1], rowl[:, :_M - 1]], axis=1)
            newg = jnp.where(lane8 < p, rowg, jnp.where(lane8 == p, v, shg))
            newloc = jnp.int32(cand * _L) + c
            newl = jnp.where(lane8 < p, rowl,
                             jnp.where(lane8 == p, newloc, shl))
            glob_ref[pl.ds(tb, 1), :] = jnp.where(present, rowg, newg)
            loc_ref[pl.ds(tb, 1), :] = jnp.where(present, rowl, newl)
            ba_row = jnp.where(lane32 == c, tb, ba_row)
            nbins_c = nbins_c + jnp.where(f, 0, 1).astype(i32)
            return masked, vsv, m, nbins_c, ba_row

        def _step16(k16, carry):
            for t in range(16):
                carry = _step(k16 * 16 + t, carry)
            return carry

        carry0 = (mat, jnp.zeros((1, _L), f32),
                  jnp.float32(0.0), nbins, jnp.full((1, _L), -1, i32))
        _, vsv, _, nbins, ba_row = lax.fori_loop(0, _L // 16, _step16, carry0)

        mv_ref[pl.ds(cand - 1, 1), :] = vsv
        ba_ref[pl.ds(cand, 1), :] = ba_row


def kernel(binSets, embeds, THRESHOLD):
    bins = jnp.asarray(binSets, jnp.int32)
    emb = jnp.asarray(embeds, jnp.float32)
    thr = jnp.asarray(THRESHOLD).astype(jnp.float32).reshape(1, 1)
    table = _sc_gather_fn()(emb, bins.reshape(-1))
    ba, mv = pl.pallas_call(
        _match_body,
        in_specs=[
            pl.BlockSpec(memory_space=pltpu.SMEM),
            pl.BlockSpec(memory_space=pltpu.SMEM),
            pl.BlockSpec(memory_space=pltpu.VMEM),
        ],
        out_specs=[
            pl.BlockSpec(memory_space=pltpu.VMEM),
            pl.BlockSpec(memory_space=pltpu.VMEM),
        ],
        out_shape=[
            jax.ShapeDtypeStruct((_S, _L), jnp.int32),
            jax.ShapeDtypeStruct((_S - 1, _L), jnp.float32),
        ],
        scratch_shapes=[
            pltpu.VMEM((_B, _M), jnp.int32),
            pltpu.VMEM((_B, _M), jnp.int32),
        ],
    )(bins, thr, table)
    return ba, jnp.reshape(mv, (-1,))
